# baseline (speedup 1.0000x reference)
import functools

import jax
import jax.numpy as jnp
from jax import lax
from jax.experimental import pallas as pl
from jax.experimental.pallas import tpu as pltpu, tpu_sc as plsc

B = 16384
F = 100
NC = 2
NS = 16
NW = NC * NS
RPW = B // NW
CH = 64
NCHUNK = RPW // CH
E = CH * F
GROUPS = CH // 16
UNROLL = 5

VOCAB = 1000000
TCHUNK = 5000
NTCHUNK = VOCAB // TCHUNK


def _wide_body(idx_hbm, val_hbm, tab_hbm, bias_hbm, out_hbm,
               idx_v0, val_v0, g_v0, idx_v1, val_v1, g_v1,
               out_v, bias_v, tab_sh,
               semi0, semv0, semg0, semi1, semv1, semg1):
    c = lax.axis_index("c")
    s = lax.axis_index("s")
    wid = s * NC + c

    for k in range((NTCHUNK + NS - 1) // NS):
        t = s + k * NS

        @pl.when(t < NTCHUNK)
        def _stage_one(t=t):
            off = t * TCHUNK
            pltpu.sync_copy(tab_hbm.at[pl.ds(off, TCHUNK)],
                            val_v0.at[pl.ds(0, TCHUNK)])
            pltpu.sync_copy(val_v0.at[pl.ds(0, TCHUNK)],
                            tab_sh.at[pl.ds(off, TCHUNK)])

    pltpu.sync_copy(bias_hbm, bias_v)
    riota = lax.iota(jnp.int32, 16) * F
    bias_vec = bias_v[...]
    plsc.subcore_barrier()

    bufs = [(idx_v0, val_v0, g_v0, semi0, semv0, semg0),
            (idx_v1, val_v1, g_v1, semi1, semv1, semg1)]
    idx_d, val_d, g_d = {}, {}, {}

    def start_idx(ci):
        bidx, _, _, bsemi, _, _ = bufs[ci % 2]
        base = wid * (RPW * F) + ci * E
        idx_d[ci] = pltpu.async_copy(idx_hbm.at[pl.ds(base, E)], bidx, bsemi)

    def start_gv(ci):
        bidx, bval, bg, _, bsemv, bsemg = bufs[ci % 2]
        base = wid * (RPW * F) + ci * E
        g_d[ci] = pltpu.async_copy(tab_sh.at[bidx], bg, bsemg)
        val_d[ci] = pltpu.async_copy(val_hbm.at[pl.ds(base, E)], bval, bsemv)

    start_idx(0)
    idx_d[0].wait()
    start_gv(0)
    if NCHUNK > 1:
        start_idx(1)
    for ci in range(NCHUNK):
        if ci + 1 < NCHUNK:
            idx_d[ci + 1].wait()
            start_gv(ci + 1)
        g_d[ci].wait()
        val_d[ci].wait()
        if ci + 2 < NCHUNK:
            start_idx(ci + 2)
        _, bval, bg, _, _, _ = bufs[ci % 2]
        for g in range(GROUPS):
            gbase = g * (16 * F)

            def body(it, acc, gbase=gbase, bg=bg, bval=bval):
                for d in range(UNROLL):
                    ii = riota + (gbase + it * UNROLL + d)
                    acc = acc + plsc.load_gather(bg, [ii]) * \
                        plsc.load_gather(bval, [ii])
                return acc

            acc = lax.fori_loop(0, F // UNROLL, body, bias_vec)
            out_v[pl.ds(ci * CH + g * 16, 16)] = acc
    pltpu.sync_copy(out_v, out_hbm.at[pl.ds(wid * RPW, RPW)])


@functools.partial(jax.jit, static_argnames=())
def _wide(idx, val, tab, bias16):
    mesh = plsc.VectorSubcoreMesh(core_axis_name="c", subcore_axis_name="s",
                                  num_cores=NC, num_subcores=NS)
    return pl.kernel(
        _wide_body,
        out_type=jax.ShapeDtypeStruct((B,), jnp.float32),
        mesh=mesh,
        compiler_params=pltpu.CompilerParams(needs_layout_passes=False),
        scratch_types=[
            pltpu.VMEM((E,), jnp.int32),
            pltpu.VMEM((E,), jnp.float32),
            pltpu.VMEM((E,), jnp.float32),
            pltpu.VMEM((E,), jnp.int32),
            pltpu.VMEM((E,), jnp.float32),
            pltpu.VMEM((E,), jnp.float32),
            pltpu.VMEM((RPW,), jnp.float32),
            pltpu.VMEM((16,), jnp.float32),
            pltpu.VMEM_SHARED((VOCAB,), jnp.float32),
            pltpu.SemaphoreType.DMA,
            pltpu.SemaphoreType.DMA,
            pltpu.SemaphoreType.DMA,
            pltpu.SemaphoreType.DMA,
            pltpu.SemaphoreType.DMA,
            pltpu.SemaphoreType.DMA,
        ],
    )(idx, val, tab, bias16)


def kernel(index, value, field, table, bias):
    del field
    idx = index.reshape(-1).astype(jnp.int32)
    val = value.reshape(-1).astype(jnp.float32)
    tab = jnp.ravel(table.T)
    bias16 = jnp.broadcast_to(bias.astype(jnp.float32), (16,))
    out = _wide(idx, val, tab, bias16)
    return out.reshape(B, 1)


# speedup vs baseline: 1.0025x; 1.0025x over previous
import functools

import jax
import jax.numpy as jnp
from jax import lax
from jax.experimental import pallas as pl
from jax.experimental.pallas import tpu as pltpu, tpu_sc as plsc

B = 16384
F = 100
NC = 2
NS = 16
NW = NC * NS
RPW = B // NW
CH = 64
NCHUNK = RPW // CH
E = CH * F
GROUPS = CH // 16
UNROLL = 4

VOCAB = 1000000
TCHUNK = 5000
NTCHUNK = VOCAB // TCHUNK


def _wide_body(idx_hbm, val_hbm, tab_hbm, bias_hbm, out_hbm,
               idx_v0, val_v0, g_v0, idx_v1, val_v1, g_v1,
               out_v, bias_v, tab_sh,
               semi0, semv0, semg0, semi1, semv1, semg1):
    c = lax.axis_index("c")
    s = lax.axis_index("s")
    wid = s * NC + c

    for k in range((NTCHUNK + NS - 1) // NS):
        t = s + k * NS

        @pl.when(t < NTCHUNK)
        def _stage_one(t=t):
            off = t * TCHUNK
            pltpu.sync_copy(tab_hbm.at[pl.ds(off, TCHUNK)],
                            val_v0.at[pl.ds(0, TCHUNK)])
            pltpu.sync_copy(val_v0.at[pl.ds(0, TCHUNK)],
                            tab_sh.at[pl.ds(off, TCHUNK)])

    pltpu.sync_copy(bias_hbm, bias_v)
    riota = lax.iota(jnp.int32, 16) * F
    bias_vec = bias_v[...]
    plsc.subcore_barrier()

    bufs = [(idx_v0, val_v0, g_v0, semi0, semv0, semg0),
            (idx_v1, val_v1, g_v1, semi1, semv1, semg1)]
    idx_d, val_d, g_d = {}, {}, {}

    def start_idx(ci):
        bidx, _, _, bsemi, _, _ = bufs[ci % 2]
        base = wid * (RPW * F) + ci * E
        idx_d[ci] = pltpu.async_copy(idx_hbm.at[pl.ds(base, E)], bidx, bsemi)

    def start_gv(ci):
        bidx, bval, bg, _, bsemv, bsemg = bufs[ci % 2]
        base = wid * (RPW * F) + ci * E
        g_d[ci] = pltpu.async_copy(tab_sh.at[bidx], bg, bsemg)
        val_d[ci] = pltpu.async_copy(val_hbm.at[pl.ds(base, E)], bval, bsemv)

    start_idx(0)
    idx_d[0].wait()
    start_gv(0)
    if NCHUNK > 1:
        start_idx(1)
    for ci in range(NCHUNK):
        if ci + 1 < NCHUNK:
            idx_d[ci + 1].wait()
            start_gv(ci + 1)
        g_d[ci].wait()
        val_d[ci].wait()
        if ci + 2 < NCHUNK:
            start_idx(ci + 2)
        _, bval, bg, _, _, _ = bufs[ci % 2]
        for g in range(GROUPS):
            gbase = g * (16 * F)

            def body(it, acc, gbase=gbase, bg=bg, bval=bval):
                for d in range(UNROLL):
                    ii = riota + (gbase + it * UNROLL + d)
                    acc = acc + plsc.load_gather(bg, [ii]) * \
                        plsc.load_gather(bval, [ii])
                return acc

            acc = lax.fori_loop(0, F // UNROLL, body, bias_vec)
            out_v[pl.ds(ci * CH + g * 16, 16)] = acc
    pltpu.sync_copy(out_v, out_hbm.at[pl.ds(wid * RPW, RPW)])


@functools.partial(jax.jit, static_argnames=())
def _wide(idx, val, tab, bias16):
    mesh = plsc.VectorSubcoreMesh(core_axis_name="c", subcore_axis_name="s",
                                  num_cores=NC, num_subcores=NS)
    return pl.kernel(
        _wide_body,
        out_type=jax.ShapeDtypeStruct((B,), jnp.float32),
        mesh=mesh,
        compiler_params=pltpu.CompilerParams(needs_layout_passes=False),
        scratch_types=[
            pltpu.VMEM((E,), jnp.int32),
            pltpu.VMEM((E,), jnp.float32),
            pltpu.VMEM((E,), jnp.float32),
            pltpu.VMEM((E,), jnp.int32),
            pltpu.VMEM((E,), jnp.float32),
            pltpu.VMEM((E,), jnp.float32),
            pltpu.VMEM((RPW,), jnp.float32),
            pltpu.VMEM((16,), jnp.float32),
            pltpu.VMEM_SHARED((VOCAB,), jnp.float32),
            pltpu.SemaphoreType.DMA,
            pltpu.SemaphoreType.DMA,
            pltpu.SemaphoreType.DMA,
            pltpu.SemaphoreType.DMA,
            pltpu.SemaphoreType.DMA,
            pltpu.SemaphoreType.DMA,
        ],
    )(idx, val, tab, bias16)


def kernel(index, value, field, table, bias):
    del field
    idx = index.reshape(-1).astype(jnp.int32)
    val = value.reshape(-1).astype(jnp.float32)
    tab = jnp.ravel(table.T)
    bias16 = jnp.broadcast_to(bias.astype(jnp.float32), (16,))
    out = _wide(idx, val, tab, bias16)
    return out.reshape(B, 1)
